# baseline (speedup 1.0000x reference)
import jax
import jax.numpy as jnp
from jax.experimental import pallas as pl

K = 1024
D = 256
SUB = 512


def _vq_block(lat_ref, cb_ref, out_ref, loss_ref):
    cb = cb_ref[...]
    cb2 = jnp.sum(cb * cb, axis=1)
    thw = lat_ref.shape[2]
    acc = jnp.zeros((), jnp.float32)
    for j in range(thw // SUB):
        lt = lat_ref[0, :, j * SUB:(j + 1) * SUB]
        f2 = jnp.sum(lt * lt, axis=0)[:, None]
        mm = jax.lax.dot_general(lt, cb, (((0,), (1,)), ((), ())),
                                 preferred_element_type=jnp.float32)
        dist = (f2 + cb2) - 2.0 * mm
        m = jnp.min(dist, axis=1, keepdims=True)
        iota = jax.lax.broadcasted_iota(jnp.int32, dist.shape, 1)
        idx = jnp.min(jnp.where(dist == m, iota, K), axis=1)
        oh = (iota == idx[:, None]).astype(jnp.float32)
        qt = jax.lax.dot_general(cb, oh, (((0,), (1,)), ((), ())),
                                 preferred_element_type=jnp.float32)
        dt = qt - lt
        out_ref[0, :, j * SUB:(j + 1) * SUB] = lt + dt
        acc = acc + jnp.sum(dt * dt)
    loss_ref[...] = jnp.full((1, 1, 128), acc, jnp.float32)


def kernel(latents, vq_weight, codebook):
    b, d, t, h, w = latents.shape
    thw = t * h * w
    lat3 = latents.reshape(b, d, thw)
    out3, lossp = pl.pallas_call(
        _vq_block,
        grid=(b,),
        in_specs=[pl.BlockSpec((1, D, thw), lambda i: (i, 0, 0)),
                  pl.BlockSpec((K, D), lambda i: (0, 0))],
        out_specs=[pl.BlockSpec((1, D, thw), lambda i: (i, 0, 0)),
                   pl.BlockSpec((1, 1, 128), lambda i: (i, 0, 0))],
        out_shape=[jax.ShapeDtypeStruct((b, d, thw), jnp.float32),
                   jax.ShapeDtypeStruct((b, 1, 128), jnp.float32)],
    )(lat3, codebook)
    s = jnp.sum(lossp[:, 0, 0])
    mean = s / (b * thw * d)
    vq_loss = mean * vq_weight + mean
    return out3.reshape(b, d, t, h, w), vq_loss


# speedup vs baseline: 1.0159x; 1.0159x over previous
import jax
import jax.numpy as jnp
from jax.experimental import pallas as pl

K = 1024
D = 256
NB = 1024


def _vq_block(flat_ref, lat_ref, cb_ref, out_ref, loss_ref):
    flat = flat_ref[...]
    cb = cb_ref[...]
    f2 = jnp.sum(flat * flat, axis=1, keepdims=True)
    cb2 = jnp.sum(cb * cb, axis=1)
    mm = jax.lax.dot_general(flat, cb, (((1,), (1,)), ((), ())),
                             preferred_element_type=jnp.float32)
    dist = (f2 + cb2) - 2.0 * mm
    m = jnp.min(dist, axis=1, keepdims=True)
    iota = jax.lax.broadcasted_iota(jnp.int32, dist.shape, 1)
    idx = jnp.min(jnp.where(dist == m, iota, K), axis=1)
    oh = (iota == idx[:, None]).astype(jnp.float32)
    qt = jax.lax.dot_general(cb, oh, (((0,), (1,)), ((), ())),
                             preferred_element_type=jnp.float32)
    lt = lat_ref[0]
    dt = qt - lt
    out_ref[0] = lt + dt
    loss_ref[...] = jnp.full((1, 1, 128), jnp.sum(dt * dt), jnp.float32)


def kernel(latents, vq_weight, codebook):
    b, d, t, h, w = latents.shape
    thw = t * h * w
    lat3 = latents.reshape(b, d, thw)
    lat = jnp.transpose(latents, (0, 2, 3, 4, 1))
    flat = lat.reshape(-1, D)
    n = flat.shape[0]
    nblk = n // NB
    ncol = thw // NB
    out3, lossp = pl.pallas_call(
        _vq_block,
        grid=(nblk,),
        in_specs=[pl.BlockSpec((NB, D), lambda i: (i, 0)),
                  pl.BlockSpec((1, D, NB), lambda i: (i // ncol, 0, i % ncol)),
                  pl.BlockSpec((K, D), lambda i: (0, 0))],
        out_specs=[pl.BlockSpec((1, D, NB), lambda i: (i // ncol, 0, i % ncol)),
                   pl.BlockSpec((1, 1, 128), lambda i: (i, 0, 0))],
        out_shape=[jax.ShapeDtypeStruct((b, d, thw), jnp.float32),
                   jax.ShapeDtypeStruct((nblk, 1, 128), jnp.float32)],
    )(flat, lat3, codebook)
    s = jnp.sum(lossp[:, 0, 0])
    mean = s / (n * D)
    vq_loss = mean * vq_weight + mean
    return out3.reshape(b, d, t, h, w), vq_loss


# speedup vs baseline: 1.2492x; 1.2297x over previous
import jax
import jax.numpy as jnp
from jax.experimental import pallas as pl

K = 1024
D = 256
NB = 512


def _vq_block(flat_ref, cb_ref, out_ref, loss_ref):
    flat = flat_ref[...]
    cb = cb_ref[...]
    f2 = jnp.sum(flat * flat, axis=1, keepdims=True)
    cb2 = jnp.sum(cb * cb, axis=1)
    mm = jax.lax.dot_general(flat, cb, (((1,), (1,)), ((), ())),
                             preferred_element_type=jnp.float32)
    dist = (f2 + cb2) - 2.0 * mm
    m = jnp.min(dist, axis=1, keepdims=True)
    iota = jax.lax.broadcasted_iota(jnp.int32, dist.shape, 1)
    idx = jnp.min(jnp.where(dist == m, iota, K), axis=1)
    oh = (iota == idx[:, None]).astype(jnp.float32)
    q = jax.lax.dot_general(oh, cb, (((1,), (0,)), ((), ())),
                            preferred_element_type=jnp.float32)
    diff = q - flat
    out_ref[...] = flat + diff
    loss_ref[...] = jnp.full((1, 1, 128), jnp.sum(diff * diff), jnp.float32)


def kernel(latents, vq_weight, codebook):
    lat = jnp.transpose(latents, (0, 2, 3, 4, 1))
    lat_shape = lat.shape
    flat = lat.reshape(-1, D)
    n = flat.shape[0]
    nblk = n // NB
    out, lossp = pl.pallas_call(
        _vq_block,
        grid=(nblk,),
        in_specs=[pl.BlockSpec((NB, D), lambda i: (i, 0)),
                  pl.BlockSpec((K, D), lambda i: (0, 0))],
        out_specs=[pl.BlockSpec((NB, D), lambda i: (i, 0)),
                   pl.BlockSpec((1, 1, 128), lambda i: (i, 0, 0))],
        out_shape=[jax.ShapeDtypeStruct((n, D), jnp.float32),
                   jax.ShapeDtypeStruct((nblk, 1, 128), jnp.float32)],
    )(flat, codebook)
    s = jnp.sum(lossp[:, 0, 0])
    mean = s / (n * D)
    vq_loss = mean * vq_weight + mean
    out5 = out.reshape(lat_shape)
    return jnp.transpose(out5, (0, 4, 1, 2, 3)), vq_loss


# speedup vs baseline: 1.5470x; 1.2384x over previous
import jax
import jax.numpy as jnp
from jax.experimental import pallas as pl

K = 1024
D = 256
NB = 2048


def _vq_block(flat_ref, cb_ref, out_ref, loss_ref):
    flat = flat_ref[...]
    cb = cb_ref[...]
    f2 = jnp.sum(flat * flat, axis=1, keepdims=True)
    cb2 = jnp.sum(cb * cb, axis=1)
    mm = jax.lax.dot_general(flat, cb, (((1,), (1,)), ((), ())),
                             preferred_element_type=jnp.float32)
    dist = (f2 + cb2) - 2.0 * mm
    m = jnp.min(dist, axis=1, keepdims=True)
    iota = jax.lax.broadcasted_iota(jnp.int32, dist.shape, 1)
    idx = jnp.min(jnp.where(dist == m, iota, K), axis=1)
    oh = (iota == idx[:, None]).astype(jnp.float32)
    q = jax.lax.dot_general(oh, cb, (((1,), (0,)), ((), ())),
                            preferred_element_type=jnp.float32)
    diff = q - flat
    out_ref[...] = flat + diff
    loss_ref[...] = jnp.full((1, 1, 128), jnp.sum(diff * diff), jnp.float32)


def kernel(latents, vq_weight, codebook):
    lat = jnp.transpose(latents, (0, 2, 3, 4, 1))
    lat_shape = lat.shape
    flat = lat.reshape(-1, D)
    n = flat.shape[0]
    nblk = n // NB
    out, lossp = pl.pallas_call(
        _vq_block,
        grid=(nblk,),
        in_specs=[pl.BlockSpec((NB, D), lambda i: (i, 0)),
                  pl.BlockSpec((K, D), lambda i: (0, 0))],
        out_specs=[pl.BlockSpec((NB, D), lambda i: (i, 0)),
                   pl.BlockSpec((1, 1, 128), lambda i: (i, 0, 0))],
        out_shape=[jax.ShapeDtypeStruct((n, D), jnp.float32),
                   jax.ShapeDtypeStruct((nblk, 1, 128), jnp.float32)],
    )(flat, codebook)
    s = jnp.sum(lossp[:, 0, 0])
    mean = s / (n * D)
    vq_loss = mean * vq_weight + mean
    out5 = out.reshape(lat_shape)
    return jnp.transpose(out5, (0, 4, 1, 2, 3)), vq_loss


# speedup vs baseline: 1.5618x; 1.0096x over previous
import jax
import jax.numpy as jnp
from jax.experimental import pallas as pl

K = 1024
D = 256
NB = 4096


def _vq_block(flat_ref, cb_ref, out_ref, loss_ref):
    flat = flat_ref[...]
    cb = cb_ref[...]
    f2 = jnp.sum(flat * flat, axis=1, keepdims=True)
    cb2 = jnp.sum(cb * cb, axis=1)
    mm = jax.lax.dot_general(flat, cb, (((1,), (1,)), ((), ())),
                             preferred_element_type=jnp.float32)
    dist = (f2 + cb2) - 2.0 * mm
    m = jnp.min(dist, axis=1, keepdims=True)
    iota = jax.lax.broadcasted_iota(jnp.int32, dist.shape, 1)
    idx = jnp.min(jnp.where(dist == m, iota, K), axis=1)
    oh = (iota == idx[:, None]).astype(jnp.float32)
    q = jax.lax.dot_general(oh, cb, (((1,), (0,)), ((), ())),
                            preferred_element_type=jnp.float32)
    diff = q - flat
    out_ref[...] = flat + diff
    loss_ref[...] = jnp.full((1, 1, 128), jnp.sum(diff * diff), jnp.float32)


def kernel(latents, vq_weight, codebook):
    lat = jnp.transpose(latents, (0, 2, 3, 4, 1))
    lat_shape = lat.shape
    flat = lat.reshape(-1, D)
    n = flat.shape[0]
    nblk = n // NB
    out, lossp = pl.pallas_call(
        _vq_block,
        grid=(nblk,),
        in_specs=[pl.BlockSpec((NB, D), lambda i: (i, 0)),
                  pl.BlockSpec((K, D), lambda i: (0, 0))],
        out_specs=[pl.BlockSpec((NB, D), lambda i: (i, 0)),
                   pl.BlockSpec((1, 1, 128), lambda i: (i, 0, 0))],
        out_shape=[jax.ShapeDtypeStruct((n, D), jnp.float32),
                   jax.ShapeDtypeStruct((nblk, 1, 128), jnp.float32)],
    )(flat, codebook)
    s = jnp.sum(lossp[:, 0, 0])
    mean = s / (n * D)
    vq_loss = mean * vq_weight + mean
    out5 = out.reshape(lat_shape)
    return jnp.transpose(out5, (0, 4, 1, 2, 3)), vq_loss


# speedup vs baseline: 1.7250x; 1.1045x over previous
import jax
import jax.numpy as jnp
from jax.experimental import pallas as pl

K = 1024
D = 256
NB = 4096
C = 128


def _vq_block(flat_ref, cb_ref, out_ref, loss_ref):
    flat = flat_ref[...]
    cb = cb_ref[...]
    f2 = jnp.sum(flat * flat, axis=1, keepdims=True)
    cb2 = jnp.sum(cb * cb, axis=1)
    mm = jax.lax.dot_general(flat, cb, (((1,), (1,)), ((), ())),
                             preferred_element_type=jnp.float32)
    dist = (f2 + cb2) - 2.0 * mm
    iota_c = jax.lax.broadcasted_iota(jnp.int32, (NB, C), 1)
    val = dist[:, 0:C]
    ind = iota_c
    for c in range(1, K // C):
        dc = dist[:, c * C:(c + 1) * C]
        lt = dc < val
        val = jnp.minimum(val, dc)
        ind = jnp.where(lt, iota_c + c * C, ind)
    m = jnp.min(val, axis=1, keepdims=True)
    idx = jnp.min(jnp.where(val == m, ind, K), axis=1, keepdims=True)
    iota = jax.lax.broadcasted_iota(jnp.int32, (NB, K), 1)
    oh = (iota == idx).astype(jnp.float32)
    q = jax.lax.dot_general(oh, cb, (((1,), (0,)), ((), ())),
                            preferred_element_type=jnp.float32)
    diff = q - flat
    out_ref[...] = flat + diff
    loss_ref[...] = jnp.full((1, 1, 128), jnp.sum(diff * diff), jnp.float32)


def kernel(latents, vq_weight, codebook):
    lat = jnp.transpose(latents, (0, 2, 3, 4, 1))
    lat_shape = lat.shape
    flat = lat.reshape(-1, D)
    n = flat.shape[0]
    nblk = n // NB
    out, lossp = pl.pallas_call(
        _vq_block,
        grid=(nblk,),
        in_specs=[pl.BlockSpec((NB, D), lambda i: (i, 0)),
                  pl.BlockSpec((K, D), lambda i: (0, 0))],
        out_specs=[pl.BlockSpec((NB, D), lambda i: (i, 0)),
                   pl.BlockSpec((1, 1, 128), lambda i: (i, 0, 0))],
        out_shape=[jax.ShapeDtypeStruct((n, D), jnp.float32),
                   jax.ShapeDtypeStruct((nblk, 1, 128), jnp.float32)],
    )(flat, codebook)
    s = jnp.sum(lossp[:, 0, 0])
    mean = s / (n * D)
    vq_loss = mean * vq_weight + mean
    out5 = out.reshape(lat_shape)
    return jnp.transpose(out5, (0, 4, 1, 2, 3)), vq_loss


# speedup vs baseline: 1.8519x; 1.0736x over previous
import jax
import jax.numpy as jnp
from jax.experimental import pallas as pl

K = 1024
D = 256
NB = 4096
C = 128


def _vq_block(flat_ref, cb_ref, out_ref, loss_ref):
    flat = flat_ref[...]
    cb = cb_ref[...]
    f2 = jnp.sum(flat * flat, axis=1, keepdims=True)
    cb2 = jnp.sum(cb * cb, axis=1)
    mm = jax.lax.dot_general(flat, cb, (((1,), (1,)), ((), ())),
                             preferred_element_type=jnp.float32)
    iota_cf = jax.lax.broadcasted_iota(
        jnp.int32, (NB, C), 1).astype(jnp.float32)
    val = (f2 + cb2[0:C]) - 2.0 * mm[:, 0:C]
    ind = iota_cf
    for c in range(1, K // C):
        dc = (f2 + cb2[c * C:(c + 1) * C]) - 2.0 * mm[:, c * C:(c + 1) * C]
        lt = dc < val
        val = jnp.minimum(val, dc)
        ind = jnp.where(lt, iota_cf + float(c * C), ind)
    m = jnp.min(val, axis=1, keepdims=True)
    idxf = jnp.min(jnp.where(val == m, ind, float(K)), axis=1, keepdims=True)
    idx = idxf.astype(jnp.int32)
    iota = jax.lax.broadcasted_iota(jnp.int32, (NB, K), 1)
    oh = (iota == idx).astype(jnp.bfloat16)
    q = jax.lax.dot_general(oh, cb, (((1,), (0,)), ((), ())),
                            preferred_element_type=jnp.float32)
    diff = q - flat
    out_ref[...] = flat + diff
    loss_ref[...] = jnp.full((1, 1, 128), jnp.sum(diff * diff), jnp.float32)


def kernel(latents, vq_weight, codebook):
    lat = jnp.transpose(latents, (0, 2, 3, 4, 1))
    lat_shape = lat.shape
    flat = lat.reshape(-1, D)
    n = flat.shape[0]
    nblk = n // NB
    out, lossp = pl.pallas_call(
        _vq_block,
        grid=(nblk,),
        in_specs=[pl.BlockSpec((NB, D), lambda i: (i, 0)),
                  pl.BlockSpec((K, D), lambda i: (0, 0))],
        out_specs=[pl.BlockSpec((NB, D), lambda i: (i, 0)),
                   pl.BlockSpec((1, 1, 128), lambda i: (i, 0, 0))],
        out_shape=[jax.ShapeDtypeStruct((n, D), jnp.float32),
                   jax.ShapeDtypeStruct((nblk, 1, 128), jnp.float32)],
    )(flat, codebook)
    s = jnp.sum(lossp[:, 0, 0])
    mean = s / (n * D)
    vq_loss = mean * vq_weight + mean
    out5 = out.reshape(lat_shape)
    return jnp.transpose(out5, (0, 4, 1, 2, 3)), vq_loss
